# baseline (speedup 1.0000x reference)
import functools

import jax
import jax.numpy as jnp
from jax import lax
from jax.experimental import pallas as pl
from jax.experimental.pallas import tpu as pltpu
from jax.experimental.pallas import tpu_sc as plsc

NC = 2
NS = 16
NW = NC * NS

CHUNK = 64
NBUF = 4
TILE = 128
DPAD = 384


def _gather_body(idx_hbm, out_hbm, w_hbm, wt_hbm, idx_v, bufs, gsems,
                 wsems, wid, nchunks):
    base = pl.multiple_of(wid * nchunks * CHUNK, CHUNK)

    pltpu.sync_copy(idx_hbm.at[wid], idx_v)

    def start_gathers(c, buf, sem):
        idx = idx_v.at[c]
        g0 = pltpu.async_copy(w_hbm.at[idx, pl.ds(0, TILE)],
                              buf.at[:, pl.ds(0, TILE)], sem)
        g1 = pltpu.async_copy(w_hbm.at[idx, pl.ds(TILE, TILE)],
                              buf.at[:, pl.ds(TILE, TILE)], sem)
        g2 = pltpu.async_copy(wt_hbm.at[idx],
                              buf.at[:, pl.ds(2 * TILE, TILE)], sem)
        return (g0, g1, g2)

    out_main, out_tail = out_hbm

    def step(i, _):
        gs = [start_gathers(NBUF * i + j, bufs[j], gsems[j])
              for j in range(NBUF)]
        ws = []
        for j in range(NBUF):
            for g in gs[j]:
                g.wait()
            off = pl.multiple_of((NBUF * i + j) * CHUNK, CHUNK)
            ws.append(pltpu.async_copy(
                bufs[j].at[:, pl.ds(0, 2 * TILE)],
                out_main.at[pl.ds(base + off, CHUNK)], wsems[j]))
            ws.append(pltpu.async_copy(
                bufs[j].at[:, pl.ds(2 * TILE, TILE)],
                out_tail.at[pl.ds(base + off, CHUNK)], wsems[j]))
        for w in ws:
            w.wait()
        return 0

    lax.fori_loop(0, nchunks // NBUF, step, 0)


def _make_sc_gather(n_total):
    n_per_w = n_total // NW
    nchunks = n_per_w // CHUNK
    mesh = plsc.VectorSubcoreMesh(core_axis_name="c", subcore_axis_name="s")

    @functools.partial(
        pl.kernel,
        out_type=[jax.ShapeDtypeStruct((n_total, 2 * TILE), jnp.float32),
                  jax.ShapeDtypeStruct((n_total, TILE), jnp.float32)],
        mesh=mesh,
        scratch_types=(
            [pltpu.VMEM((nchunks, CHUNK), jnp.int32)]
            + [pltpu.VMEM((CHUNK, DPAD), jnp.float32)] * NBUF
            + [pltpu.SemaphoreType.DMA] * (2 * NBUF)
        ),
    )
    def sc_gather(idx_hbm, w_hbm, wt_hbm, out_main, out_tail, idx_v, *rest):
        bufs = rest[:NBUF]
        gsems = rest[NBUF:2 * NBUF]
        wsems = rest[2 * NBUF:3 * NBUF]
        wid = lax.axis_index("s") * NC + lax.axis_index("c")
        _gather_body(idx_hbm, (out_main, out_tail), w_hbm, wt_hbm, idx_v,
                     bufs, gsems, wsems, wid, nchunks)

    return sc_gather


def kernel(target_word, synonym, antonym, W):
    b, l = target_word.shape
    dim = W.shape[1]
    n = b * l
    nchunks = n // NW // CHUNK
    w_tail = jnp.pad(W[:, 2 * TILE:], ((0, 0), (0, 3 * TILE - dim)))

    fn = _make_sc_gather(n)
    outs = []
    for idx in (target_word, synonym, antonym):
        idx3 = idx.reshape(NW, nchunks, CHUNK).astype(jnp.int32)
        om, ot = fn(idx3, W, w_tail)
        o = jnp.concatenate([om, ot[:, :dim - 2 * TILE]], axis=1)
        outs.append(o.reshape(b, l, dim))
    return tuple(outs)


# speedup vs baseline: 1.2937x; 1.2937x over previous
import functools

import jax
import jax.numpy as jnp
from jax import lax
from jax.experimental import pallas as pl
from jax.experimental.pallas import tpu as pltpu
from jax.experimental.pallas import tpu_sc as plsc

NC = 2
NS = 16
NW = NC * NS

CHUNK = 64
NBUF = 4
TILE = 128
DPAD = 384


def _gather_body(idx_hbm, out_hbm, w_hbm, wt_hbm, idx_v, bufs, gsems,
                 wsems, wid, nchunks):
    base = pl.multiple_of(wid * nchunks * CHUNK, CHUNK)

    pltpu.sync_copy(idx_hbm.at[wid], idx_v)

    def start_gathers(c, buf, sem):
        idx = idx_v.at[c]
        g0 = pltpu.async_copy(w_hbm.at[idx, pl.ds(0, TILE)],
                              buf.at[:, pl.ds(0, TILE)], sem)
        g1 = pltpu.async_copy(w_hbm.at[idx, pl.ds(TILE, TILE)],
                              buf.at[:, pl.ds(TILE, TILE)], sem)
        g2 = pltpu.async_copy(wt_hbm.at[idx],
                              buf.at[:, pl.ds(2 * TILE, TILE)], sem)
        return (g0, g1, g2)

    def step(i, _):
        gs = [start_gathers(NBUF * i + j, bufs[j], gsems[j])
              for j in range(NBUF)]
        ws = []
        for j in range(NBUF):
            for g in gs[j]:
                g.wait()
            off = pl.multiple_of((NBUF * i + j) * CHUNK, CHUNK)
            ws.append(pltpu.async_copy(
                bufs[j], out_hbm.at[pl.ds(base + off, CHUNK)], wsems[j]))
        for w in ws:
            w.wait()
        return 0

    lax.fori_loop(0, nchunks // NBUF, step, 0)


def _make_sc_gather(n_total):
    n_per_w = n_total // NW
    nchunks = n_per_w // CHUNK
    mesh = plsc.VectorSubcoreMesh(core_axis_name="c", subcore_axis_name="s")

    @functools.partial(
        pl.kernel,
        out_type=jax.ShapeDtypeStruct((n_total, DPAD), jnp.float32),
        mesh=mesh,
        scratch_types=(
            [pltpu.VMEM((nchunks, CHUNK), jnp.int32)]
            + [pltpu.VMEM((CHUNK, DPAD), jnp.float32)] * NBUF
            + [pltpu.SemaphoreType.DMA] * (2 * NBUF)
        ),
    )
    def sc_gather(idx_hbm, w_hbm, wt_hbm, out_hbm, idx_v, *rest):
        bufs = rest[:NBUF]
        gsems = rest[NBUF:2 * NBUF]
        wsems = rest[2 * NBUF:3 * NBUF]
        wid = lax.axis_index("s") * NC + lax.axis_index("c")
        _gather_body(idx_hbm, out_hbm, w_hbm, wt_hbm, idx_v, bufs,
                     gsems, wsems, wid, nchunks)

    return sc_gather


def kernel(target_word, synonym, antonym, W):
    b, l = target_word.shape
    dim = W.shape[1]
    n = b * l
    nchunks = n // NW // CHUNK
    w_tail = jnp.pad(W[:, 2 * TILE:], ((0, 0), (0, 3 * TILE - dim)))

    fn = _make_sc_gather(n)
    outs = []
    for idx in (target_word, synonym, antonym):
        idx3 = idx.reshape(NW, nchunks, CHUNK).astype(jnp.int32)
        outs.append(fn(idx3, W, w_tail)[:, :dim].reshape(b, l, dim))
    return tuple(outs)
